# speedup vs baseline: 1.5380x; 1.5380x over previous
import functools

import jax
import jax.numpy as jnp
from jax import lax
from jax.experimental import pallas as pl
from jax.experimental.pallas import tpu as pltpu
from jax.experimental.pallas import tpu_sc as plsc

D_VOCAB = 50257
D_MODEL = 1024
NC = 2
NS = 16
NW = NC * NS

CHUNK = 64
N_CHUNKS = 8


def _make_embed_kernel(n_tokens):
    assert n_tokens % NW == 0
    per_w = n_tokens // NW
    assert per_w == N_CHUNKS * CHUNK

    mesh = plsc.VectorSubcoreMesh(core_axis_name="c", subcore_axis_name="s")

    @functools.partial(
        pl.kernel,
        out_type=jax.ShapeDtypeStruct((n_tokens, D_MODEL), jnp.float32),
        mesh=mesh,
        scratch_types=[
            pltpu.VMEM((N_CHUNKS, CHUNK), jnp.int32),
            pltpu.VMEM((CHUNK, D_MODEL), jnp.float32),
            pltpu.SemaphoreType.DMA,
        ],
    )
    def embed(tokens_hbm, table_hbm, out_hbm, idx_v, rows_v, sem):
        wid = lax.axis_index("s") * NC + lax.axis_index("c")
        pltpu.sync_copy(tokens_hbm.at[wid], idx_v)
        base = wid * per_w
        for c in range(N_CHUNKS):
            pltpu.async_copy(table_hbm.at[idx_v.at[c]], rows_v, sem).wait()
            pltpu.sync_copy(rows_v, out_hbm.at[pl.ds(base + c * CHUNK, CHUNK)])

    return embed


@jax.jit
def kernel(tokens, W_E):
    batch, seq = tokens.shape
    n_tokens = batch * seq
    tok3 = tokens.reshape(NW, N_CHUNKS, CHUNK).astype(jnp.int32)
    out = _make_embed_kernel(n_tokens)(tok3, W_E)
    return out.reshape(batch, seq, D_MODEL)


# speedup vs baseline: 1.6118x; 1.0480x over previous
import functools

import jax
import jax.numpy as jnp
from jax import lax
from jax.experimental import pallas as pl
from jax.experimental.pallas import tpu as pltpu
from jax.experimental.pallas import tpu_sc as plsc

D_VOCAB = 50257
D_MODEL = 1024
NC = 2
NS = 16
NW = NC * NS

CHUNK = 32
N_CHUNKS = 16


def _make_embed_kernel(n_tokens):
    assert n_tokens % NW == 0
    per_w = n_tokens // NW
    assert per_w == N_CHUNKS * CHUNK

    mesh = plsc.VectorSubcoreMesh(core_axis_name="c", subcore_axis_name="s")

    @functools.partial(
        pl.kernel,
        out_type=jax.ShapeDtypeStruct((n_tokens, D_MODEL), jnp.float32),
        mesh=mesh,
        scratch_types=[
            pltpu.VMEM((N_CHUNKS, CHUNK), jnp.int32),
            pltpu.VMEM((2, CHUNK, D_MODEL), jnp.float32),
            pltpu.SemaphoreType.DMA,
            pltpu.SemaphoreType.DMA,
            pltpu.SemaphoreType.DMA,
            pltpu.SemaphoreType.DMA,
        ],
    )
    def embed(tokens_hbm, table_hbm, out_hbm, idx_v, rows_v,
              gsem0, gsem1, wsem0, wsem1):
        gsem = (gsem0, gsem1)
        wsem = (wsem0, wsem1)
        wid = lax.axis_index("s") * NC + lax.axis_index("c")
        pltpu.sync_copy(tokens_hbm.at[wid], idx_v)
        base = wid * per_w

        def gather(c):
            return pltpu.async_copy(
                table_hbm.at[idx_v.at[c]], rows_v.at[c % 2], gsem[c % 2])

        def write(c):
            return pltpu.async_copy(
                rows_v.at[c % 2],
                out_hbm.at[pl.ds(base + c * CHUNK, CHUNK)], wsem[c % 2])

        gd = [None] * N_CHUNKS
        wd = [None] * N_CHUNKS
        gd[0] = gather(0)
        for c in range(N_CHUNKS):
            if c + 1 < N_CHUNKS:
                if c >= 1:
                    wd[c - 1].wait()
                gd[c + 1] = gather(c + 1)
            gd[c].wait()
            wd[c] = write(c)
        wd[N_CHUNKS - 1].wait()

    return embed


@jax.jit
def kernel(tokens, W_E):
    batch, seq = tokens.shape
    n_tokens = batch * seq
    tok3 = tokens.reshape(NW, N_CHUNKS, CHUNK).astype(jnp.int32)
    out = _make_embed_kernel(n_tokens)(tok3, W_E)
    return out.reshape(batch, seq, D_MODEL)


# speedup vs baseline: 1.6183x; 1.0040x over previous
import functools

import jax
import jax.numpy as jnp
from jax import lax
from jax.experimental import pallas as pl
from jax.experimental.pallas import tpu as pltpu
from jax.experimental.pallas import tpu_sc as plsc

D_VOCAB = 50257
D_MODEL = 1024
NC = 2
NS = 16
NW = NC * NS

PER_W = 512
SIZES = [64, 56, 64, 56, 64, 56, 64, 56, 32]
OFFS = [0, 64, 120, 184, 240, 304, 360, 424, 480]
NCH = len(SIZES)


def _make_embed_kernel(batch, seq):
    n_tokens = batch * seq
    assert n_tokens == NW * PER_W
    w_per_row = seq // PER_W

    mesh = plsc.VectorSubcoreMesh(core_axis_name="c", subcore_axis_name="s")

    @functools.partial(
        pl.kernel,
        out_type=jax.ShapeDtypeStruct((n_tokens, D_MODEL), jnp.float32),
        mesh=mesh,
        scratch_types=[
            pltpu.VMEM((PER_W,), jnp.int32),
            pltpu.VMEM((SIZES[0], D_MODEL), jnp.float32),
            pltpu.VMEM((SIZES[1], D_MODEL), jnp.float32),
            pltpu.SemaphoreType.DMA,
            pltpu.SemaphoreType.DMA,
            pltpu.SemaphoreType.DMA,
            pltpu.SemaphoreType.DMA,
        ],
    )
    def embed(tokens_hbm, table_hbm, out_hbm, idx_v, rows_a, rows_b,
              gsem0, gsem1, wsem0, wsem1):
        rows = (rows_a, rows_b)
        gsem = (gsem0, gsem1)
        wsem = (wsem0, wsem1)
        wid = lax.axis_index("s") * NC + lax.axis_index("c")
        b = wid // w_per_row
        off = (wid % w_per_row) * PER_W
        pltpu.sync_copy(tokens_hbm.at[b, pl.ds(off, PER_W)], idx_v)
        base = wid * PER_W

        def gather(c):
            return pltpu.async_copy(
                table_hbm.at[idx_v.at[pl.ds(OFFS[c], SIZES[c])]],
                rows[c % 2].at[pl.ds(0, SIZES[c])], gsem[c % 2])

        def write(c):
            return pltpu.async_copy(
                rows[c % 2].at[pl.ds(0, SIZES[c])],
                out_hbm.at[pl.ds(base + OFFS[c], SIZES[c])], wsem[c % 2])

        gd = [None] * NCH
        wd = [None] * NCH
        gd[0] = gather(0)
        for c in range(NCH):
            if c + 1 < NCH:
                if c >= 1:
                    wd[c - 1].wait()
                gd[c + 1] = gather(c + 1)
            gd[c].wait()
            wd[c] = write(c)
        wd[NCH - 1].wait()

    return embed


@jax.jit
def kernel(tokens, W_E):
    batch, seq = tokens.shape
    out = _make_embed_kernel(batch, seq)(tokens.astype(jnp.int32), W_E)
    return out.reshape(batch, seq, D_MODEL)


# speedup vs baseline: 1.6417x; 1.0145x over previous
import functools

import jax
import jax.numpy as jnp
from jax import lax
from jax.experimental import pallas as pl
from jax.experimental.pallas import tpu as pltpu
from jax.experimental.pallas import tpu_sc as plsc

D_VOCAB = 50257
D_MODEL = 1024
NC = 2
NS = 16
NW = NC * NS

PER_W = 512
SIZES = [40] * 12 + [32]
OFFS = [40 * i for i in range(13)]
NCH = len(SIZES)


def _make_embed_kernel(batch, seq):
    n_tokens = batch * seq
    assert n_tokens == NW * PER_W
    w_per_row = seq // PER_W

    mesh = plsc.VectorSubcoreMesh(core_axis_name="c", subcore_axis_name="s")

    @functools.partial(
        pl.kernel,
        out_type=jax.ShapeDtypeStruct((n_tokens, D_MODEL), jnp.float32),
        mesh=mesh,
        scratch_types=[
            pltpu.VMEM((PER_W,), jnp.int32),
            pltpu.VMEM((SIZES[0], D_MODEL), jnp.float32),
            pltpu.VMEM((SIZES[0], D_MODEL), jnp.float32),
            pltpu.VMEM((SIZES[0], D_MODEL), jnp.float32),
            pltpu.SemaphoreType.DMA,
            pltpu.SemaphoreType.DMA,
            pltpu.SemaphoreType.DMA,
            pltpu.SemaphoreType.DMA,
            pltpu.SemaphoreType.DMA,
            pltpu.SemaphoreType.DMA,
        ],
    )
    def embed(tokens_hbm, table_hbm, out_hbm, idx_v, rows_a, rows_b, rows_c,
              gsem0, gsem1, gsem2, wsem0, wsem1, wsem2):
        rows = (rows_a, rows_b, rows_c)
        gsem = (gsem0, gsem1, gsem2)
        wsem = (wsem0, wsem1, wsem2)
        wid = lax.axis_index("s") * NC + lax.axis_index("c")
        b = wid // w_per_row
        off = (wid % w_per_row) * PER_W
        pltpu.sync_copy(tokens_hbm.at[b, pl.ds(off, PER_W)], idx_v)
        base = wid * PER_W

        def gather(c):
            return pltpu.async_copy(
                table_hbm.at[idx_v.at[pl.ds(OFFS[c], SIZES[c])]],
                rows[c % 3].at[pl.ds(0, SIZES[c])], gsem[c % 3])

        def write(c):
            return pltpu.async_copy(
                rows[c % 3].at[pl.ds(0, SIZES[c])],
                out_hbm.at[pl.ds(base + OFFS[c], SIZES[c])], wsem[c % 3])

        gd = [None] * NCH
        wd = [None] * NCH
        gd[0] = gather(0)
        gd[1] = gather(1)
        for c in range(NCH):
            gd[c].wait()
            wd[c] = write(c)
            if c >= 1:
                wd[c - 1].wait()
            if c + 2 < NCH:
                gd[c + 2] = gather(c + 2)
        wd[NCH - 1].wait()

    return embed


@jax.jit
def kernel(tokens, W_E):
    batch, seq = tokens.shape
    out = _make_embed_kernel(batch, seq)(tokens.astype(jnp.int32), W_E)
    return out.reshape(batch, seq, D_MODEL)
